# Initial kernel scaffold; baseline (speedup 1.0000x reference)
import jax
import jax.numpy as jnp
from jax.experimental import pallas as pl


def kernel(h, X_0, X_1, t_ij, r_ij_0, r_ij_1, r_ij_2, edge_index, params):
    raise NotImplementedError("write your pallas kernel here")


# speedup vs baseline: 1.0002x; 1.0002x over previous
import jax
import jax.numpy as jnp
from jax.experimental import pallas as pl

N = 10000
E = 160000
D = 128
ERD = 64
DEG = 2
H = 8
S = 1 + 2 * DEG
CUT = 5.0


def _mlp(p, x):
    return jax.nn.silu(x @ p['W1'] + p['b1']) @ p['W2'] + p['b2']


def _layer_norm(x, g, b, eps=1e-5):
    mu = x.mean(-1, keepdims=True)
    var = ((x - mu) ** 2).mean(-1, keepdims=True)
    return (x - mu) / jnp.sqrt(var + eps) * g + b


def _cos_cutoff(d, c=CUT):
    mask = (d <= c).astype(d.dtype)
    return ((jnp.cos(jnp.pi * d / c) + 1.0) / 2.0) * mask


def _scatter_softmax(a, seg, num_segments):
    m = jax.ops.segment_max(a, seg, num_segments=num_segments)
    m = jnp.where(jnp.isfinite(m), m, 0.0)
    e = jnp.exp(a - m[seg])
    s = jax.ops.segment_sum(e, seg, num_segments=num_segments)
    return e / (s[seg] + 1e-16)


def _add_body(a_ref, b_ref, o_ref):
    o_ref[...] = a_ref[...] + b_ref[...]


def _pl_add(a, b):
    return pl.pallas_call(
        _add_body,
        out_shape=jax.ShapeDtypeStruct(a.shape, a.dtype),
    )(a, b)


def kernel(h, X_0, X_1, t_ij, r_ij_0, r_ij_1, r_ij_2, edge_index, params):
    n_j, n_i = edge_index[0], edge_index[1]
    X = [X_0, X_1]
    r = [r_ij_0, r_ij_1, r_ij_2]
    Xcat = jnp.concatenate(X, axis=1)
    eq_i = (Xcat @ params['w_vq'])[n_i]
    ek = jnp.concatenate([X[i] @ params['w_vk'][i] for i in range(DEG)], axis=1)
    ek_j = ek[n_j]
    w_ij = (eq_i * ek_j).sum(axis=1)
    dt_ij = _mlp(params['mlp_w'], w_ij) * _mlp(params['mlp_t'], t_ij)
    t_new = t_ij + dt_ij
    hn = _layer_norm(h, params['ln_g'], params['ln_b'])
    q_i = (hn @ params['w_q']).reshape(h.shape[0], H, -1)[n_i]
    k_j = (hn @ params['w_k']).reshape(h.shape[0], H, -1)[n_j]
    v_j = _mlp(params['mlp_v'], hn).reshape(h.shape[0], H, -1)[n_j]
    re = jax.nn.silu((t_new @ params['w_re']).reshape(t_new.shape[0], H, -1))
    a_ij = (q_i * k_j * re).sum(axis=-1, keepdims=True)
    sea_ij = _scatter_softmax(a_ij, n_i, h.shape[0]) * v_j
    sea_ij = sea_ij.reshape(sea_ij.shape[0], -1)
    o_ij = sea_ij + (t_new @ params['w_rs']) * _mlp(params['mlp_s'], h)[n_j] * _cos_cutoff(r[0])
    o_ij = o_ij[:, None, :]
    chunks = [o_ij[..., i * D:(i + 1) * D] for i in range(S)]
    o_s = chunks[0][:, 0, :]
    dh = jax.ops.segment_sum(o_s, n_i, num_segments=h.shape[0])
    h_new = _pl_add(h, dh)
    X_new = []
    for i in range(DEG):
        l = i + 1
        dX = jax.ops.segment_sum(chunks[1 + i] * r[l][..., None]
                                 + chunks[1 + DEG + i] * X[i][n_j],
                                 n_i, num_segments=h.shape[0])
        X_new.append(X[i] + dX)
    return (h_new, X_new[0], X_new[1], t_new)


# speedup vs baseline: 3.6888x; 3.6881x over previous
import jax
import jax.numpy as jnp
from jax.experimental import pallas as pl

N = 10000
E = 160000
D = 128
ERD = 64
DEG = 2
H = 8
S = 1 + 2 * DEG
CUT = 5.0


def _mlp(p, x):
    return jax.nn.silu(x @ p['W1'] + p['b1']) @ p['W2'] + p['b2']


def _layer_norm(x, g, b, eps=1e-5):
    mu = x.mean(-1, keepdims=True)
    var = ((x - mu) ** 2).mean(-1, keepdims=True)
    return (x - mu) / jnp.sqrt(var + eps) * g + b


def _cos_cutoff(d, c=CUT):
    mask = (d <= c).astype(d.dtype)
    return ((jnp.cos(jnp.pi * d / c) + 1.0) / 2.0) * mask


def _scatter_softmax(a, seg, num_segments):
    m = jax.ops.segment_max(a, seg, num_segments=num_segments)
    m = jnp.where(jnp.isfinite(m), m, 0.0)
    e = jnp.exp(a - m[seg])
    s = jax.ops.segment_sum(e, seg, num_segments=num_segments)
    return e / (s[seg] + 1e-16)


def _add_body(a_ref, b_ref, o_ref):
    o_ref[...] = a_ref[...] + b_ref[...]


def _pl_add(a, b):
    return pl.pallas_call(
        _add_body,
        out_shape=jax.ShapeDtypeStruct(a.shape, a.dtype),
    )(a, b)


def kernel(h, X_0, X_1, t_ij, r_ij_0, r_ij_1, r_ij_2, edge_index, params):
    n_j, n_i = edge_index[0], edge_index[1]
    X = [X_0, X_1]
    r = [r_ij_0, r_ij_1, r_ij_2]
    Xcat = jnp.concatenate(X, axis=1)
    eq_i = (Xcat @ params['w_vq'])[n_i]
    ek = jnp.concatenate([X[i] @ params['w_vk'][i] for i in range(DEG)], axis=1)
    ek_j = ek[n_j]
    w_ij = (eq_i * ek_j).sum(axis=1)
    dt_ij = _mlp(params['mlp_w'], w_ij) * _mlp(params['mlp_t'], t_ij)
    t_new = t_ij + dt_ij
    hn = _layer_norm(h, params['ln_g'], params['ln_b'])
    q_i = (hn @ params['w_q']).reshape(h.shape[0], H, -1)[n_i]
    k_j = (hn @ params['w_k']).reshape(h.shape[0], H, -1)[n_j]
    v_j = _mlp(params['mlp_v'], hn).reshape(h.shape[0], H, -1)[n_j]
    re = jax.nn.silu((t_new @ params['w_re']).reshape(t_new.shape[0], H, -1))
    a_ij = (q_i * k_j * re).sum(axis=-1, keepdims=True)
    sea_ij = _scatter_softmax(a_ij, n_i, h.shape[0]) * v_j
    sea_ij = sea_ij.reshape(sea_ij.shape[0], -1)
    o_ij = sea_ij + (t_new @ params['w_rs']) * _mlp(params['mlp_s'], h)[n_j] * _cos_cutoff(r[0])
    chunks = [o_ij[:, i * D:(i + 1) * D] for i in range(S)]
    u0 = chunks[0]
    u1 = (chunks[1][:, None, :] * r[1][..., None]
          + chunks[3][:, None, :] * X[0][n_j]).reshape(E, 3 * D)
    u2 = (chunks[2][:, None, :] * r[2][..., None]
          + chunks[4][:, None, :] * X[1][n_j]).reshape(E, 5 * D)
    upd = jnp.concatenate([u0, u1, u2], axis=1)
    agg = jax.ops.segment_sum(upd, n_i, num_segments=h.shape[0])
    h_new = _pl_add(h, agg[:, :D])
    X_new = [X[0] + agg[:, D:4 * D].reshape(N, 3, D),
             X[1] + agg[:, 4 * D:].reshape(N, 5, D)]
    return (h_new, X_new[0], X_new[1], t_new)


# speedup vs baseline: 4.7793x; 1.2956x over previous
import functools

import jax
import jax.numpy as jnp
from jax import lax
from jax.experimental import pallas as pl
from jax.experimental.pallas import tpu as pltpu
from jax.experimental.pallas import tpu_sc as plsc

N = 10000
E = 160000
D = 128
ERD = 64
DEG = 2
H = 8
S = 1 + 2 * DEG
CUT = 5.0


def _mlp(p, x):
    return jax.nn.silu(x @ p['W1'] + p['b1']) @ p['W2'] + p['b2']


def _layer_norm(x, g, b, eps=1e-5):
    mu = x.mean(-1, keepdims=True)
    var = ((x - mu) ** 2).mean(-1, keepdims=True)
    return (x - mu) / jnp.sqrt(var + eps) * g + b


def _cos_cutoff(d, c=CUT):
    mask = (d <= c).astype(d.dtype)
    return ((jnp.cos(jnp.pi * d / c) + 1.0) / 2.0) * mask


def _scatter_softmax(a, seg, num_segments):
    m = jax.ops.segment_max(a, seg, num_segments=num_segments)
    m = jnp.where(jnp.isfinite(m), m, 0.0)
    e = jnp.exp(a - m[seg])
    s = jax.ops.segment_sum(e, seg, num_segments=num_segments)
    return e / (s[seg] + 1e-16)


_NW = 32
_EPW = E // _NW
_C = 40
_NCH = _EPW // _C


def _gather_wqk(Ta, Tb, ni_arr, nj_arr):
    mesh = plsc.VectorSubcoreMesh(core_axis_name="c", subcore_axis_name="s")

    @functools.partial(
        pl.kernel,
        out_type=(jax.ShapeDtypeStruct((E, ERD), jnp.float32),
                  jax.ShapeDtypeStruct((E, D), jnp.float32)),
        mesh=mesh,
        scratch_types=[
            pltpu.VMEM((_C,), jnp.int32),
            pltpu.VMEM((_C,), jnp.int32),
            pltpu.VMEM((_C, 640), jnp.float32),
            pltpu.VMEM((_C, 640), jnp.float32),
            pltpu.VMEM((_C, ERD), jnp.float32),
            pltpu.VMEM((_C, D), jnp.float32),
            pltpu.SemaphoreType.DMA,
            pltpu.SemaphoreType.DMA,
        ],
    )
    def k(ta_hbm, tb_hbm, ni_hbm, nj_hbm, w_hbm, qk_hbm,
          idx_i, idx_j, arows, brows, wst, qkst, sem1, sem2):
        wid = lax.axis_index("s") * 2 + lax.axis_index("c")

        def chunk_body(ch, carry):
            base = wid * _EPW + ch * _C
            pltpu.sync_copy(ni_hbm.at[pl.ds(base, _C)], idx_i)
            pltpu.sync_copy(nj_hbm.at[pl.ds(base, _C)], idx_j)
            cp1 = pltpu.async_copy(ta_hbm.at[idx_i], arows, sem1)
            cp2 = pltpu.async_copy(tb_hbm.at[idx_j], brows, sem2)
            cp1.wait()
            cp2.wait()

            def edge_body(e, c2):
                for g in range(4):
                    acc = arows[e, pl.ds(g * 16, 16)] * brows[e, pl.ds(g * 16, 16)]
                    for s_ in range(1, 8):
                        off = s_ * ERD + g * 16
                        acc = acc + (arows[e, pl.ds(off, 16)]
                                     * brows[e, pl.ds(off, 16)])
                    wst[e, pl.ds(g * 16, 16)] = acc
                for g in range(8):
                    off = 512 + g * 16
                    qkst[e, pl.ds(g * 16, 16)] = (arows[e, pl.ds(off, 16)]
                                                  * brows[e, pl.ds(off, 16)])
                return c2

            lax.fori_loop(0, _C, edge_body, 0)
            pltpu.sync_copy(wst, w_hbm.at[pl.ds(base, _C)])
            pltpu.sync_copy(qkst, qk_hbm.at[pl.ds(base, _C)])
            return carry

        lax.fori_loop(0, _NCH, chunk_body, 0)

    return k(Ta, Tb, ni_arr, nj_arr)


def _add_body(a_ref, b_ref, o_ref):
    o_ref[...] = a_ref[...] + b_ref[...]


def _pl_add(a, b):
    return pl.pallas_call(
        _add_body,
        out_shape=jax.ShapeDtypeStruct(a.shape, a.dtype),
    )(a, b)


def kernel(h, X_0, X_1, t_ij, r_ij_0, r_ij_1, r_ij_2, edge_index, params):
    n_j, n_i = edge_index[0], edge_index[1]
    X = [X_0, X_1]
    r = [r_ij_0, r_ij_1, r_ij_2]
    Xcat = jnp.concatenate(X, axis=1)
    eqf = (Xcat @ params['w_vq']).reshape(N, 8 * ERD)
    ekf = jnp.concatenate(
        [X[i] @ params['w_vk'][i] for i in range(DEG)], axis=1).reshape(N, 8 * ERD)
    hn = _layer_norm(h, params['ln_g'], params['ln_b'])
    q = hn @ params['w_q']
    kk = hn @ params['w_k']
    Ta = jnp.concatenate([eqf, q], axis=1)
    Tb = jnp.concatenate([ekf, kk], axis=1)
    w_ij, qk = _gather_wqk(Ta, Tb, n_i, n_j)
    dt_ij = _mlp(params['mlp_w'], w_ij) * _mlp(params['mlp_t'], t_ij)
    t_new = t_ij + dt_ij
    v_j = _mlp(params['mlp_v'], hn).reshape(h.shape[0], H, -1)[n_j]
    re = jax.nn.silu((t_new @ params['w_re']).reshape(t_new.shape[0], H, -1))
    a_ij = (qk.reshape(E, H, -1) * re).sum(axis=-1, keepdims=True)
    sea_ij = _scatter_softmax(a_ij, n_i, h.shape[0]) * v_j
    sea_ij = sea_ij.reshape(sea_ij.shape[0], -1)
    o_ij = sea_ij + (t_new @ params['w_rs']) * _mlp(params['mlp_s'], h)[n_j] * _cos_cutoff(r[0])
    chunks = [o_ij[:, i * D:(i + 1) * D] for i in range(S)]
    u0 = chunks[0]
    u1 = (chunks[1][:, None, :] * r[1][..., None]
          + chunks[3][:, None, :] * X[0][n_j]).reshape(E, 3 * D)
    u2 = (chunks[2][:, None, :] * r[2][..., None]
          + chunks[4][:, None, :] * X[1][n_j]).reshape(E, 5 * D)
    upd = jnp.concatenate([u0, u1, u2], axis=1)
    agg = jax.ops.segment_sum(upd, n_i, num_segments=h.shape[0])
    h_new = _pl_add(h, agg[:, :D])
    X_new = [X[0] + agg[:, D:4 * D].reshape(N, 3, D),
             X[1] + agg[:, 4 * D:].reshape(N, 5, D)]
    return (h_new, X_new[0], X_new[1], t_new)


# speedup vs baseline: 5.9752x; 1.2502x over previous
import functools

import jax
import jax.numpy as jnp
from jax import lax
from jax.experimental import pallas as pl
from jax.experimental.pallas import tpu as pltpu
from jax.experimental.pallas import tpu_sc as plsc

N = 10000
E = 160000
D = 128
ERD = 64
DEG = 2
H = 8
S = 1 + 2 * DEG
CUT = 5.0


def _mlp(p, x):
    return jax.nn.silu(x @ p['W1'] + p['b1']) @ p['W2'] + p['b2']


def _layer_norm(x, g, b, eps=1e-5):
    mu = x.mean(-1, keepdims=True)
    var = ((x - mu) ** 2).mean(-1, keepdims=True)
    return (x - mu) / jnp.sqrt(var + eps) * g + b


def _cos_cutoff(d, c=CUT):
    mask = (d <= c).astype(d.dtype)
    return ((jnp.cos(jnp.pi * d / c) + 1.0) / 2.0) * mask


def _scatter_softmax(a, seg, num_segments):
    m = jax.ops.segment_max(a, seg, num_segments=num_segments)
    m = jnp.where(jnp.isfinite(m), m, 0.0)
    e = jnp.exp(a - m[seg])
    s = jax.ops.segment_sum(e, seg, num_segments=num_segments)
    return e / (s[seg] + 1e-16)


_NW = 32
_EPW = E // _NW
_C = 40
_NCH = _EPW // _C


def _gather_wqk(Ta, Tb, ni_arr, nj_arr):
    mesh = plsc.VectorSubcoreMesh(core_axis_name="c", subcore_axis_name="s")

    @functools.partial(
        pl.kernel,
        out_type=(jax.ShapeDtypeStruct((E, ERD), jnp.float32),
                  jax.ShapeDtypeStruct((E, D), jnp.float32)),
        mesh=mesh,
        scratch_types=[
            pltpu.VMEM((_C,), jnp.int32),
            pltpu.VMEM((_C,), jnp.int32),
            pltpu.VMEM((_C, 640), jnp.float32),
            pltpu.VMEM((_C, 640), jnp.float32),
            pltpu.VMEM((_C, ERD), jnp.float32),
            pltpu.VMEM((_C, D), jnp.float32),
            pltpu.SemaphoreType.DMA,
            pltpu.SemaphoreType.DMA,
        ],
    )
    def k(ta_hbm, tb_hbm, ni_hbm, nj_hbm, w_hbm, qk_hbm,
          idx_i, idx_j, arows, brows, wst, qkst, sem1, sem2):
        wid = lax.axis_index("s") * 2 + lax.axis_index("c")

        def chunk_body(ch, carry):
            base = wid * _EPW + ch * _C
            pltpu.sync_copy(ni_hbm.at[pl.ds(base, _C)], idx_i)
            pltpu.sync_copy(nj_hbm.at[pl.ds(base, _C)], idx_j)
            cp1 = pltpu.async_copy(ta_hbm.at[idx_i], arows, sem1)
            cp2 = pltpu.async_copy(tb_hbm.at[idx_j], brows, sem2)
            cp1.wait()
            cp2.wait()

            def edge_body(e, c2):
                for g in range(4):
                    acc = arows[e, pl.ds(g * 16, 16)] * brows[e, pl.ds(g * 16, 16)]
                    for s_ in range(1, 8):
                        off = s_ * ERD + g * 16
                        acc = acc + (arows[e, pl.ds(off, 16)]
                                     * brows[e, pl.ds(off, 16)])
                    wst[e, pl.ds(g * 16, 16)] = acc
                for g in range(8):
                    off = 512 + g * 16
                    qkst[e, pl.ds(g * 16, 16)] = (arows[e, pl.ds(off, 16)]
                                                  * brows[e, pl.ds(off, 16)])
                return c2

            lax.fori_loop(0, _C, edge_body, 0)
            pltpu.sync_copy(wst, w_hbm.at[pl.ds(base, _C)])
            pltpu.sync_copy(qkst, qk_hbm.at[pl.ds(base, _C)])
            return carry

        lax.fori_loop(0, _NCH, chunk_body, 0)

    return k(Ta, Tb, ni_arr, nj_arr)


_C5 = 24
_NCH5 = -(-_EPW // _C5)


def _edge_update(Tj2, rs_cc, aux, nj_arr):
    mesh = plsc.VectorSubcoreMesh(core_axis_name="c", subcore_axis_name="s")

    @functools.partial(
        pl.kernel,
        out_type=jax.ShapeDtypeStruct((E, 9 * D), jnp.float32),
        mesh=mesh,
        scratch_types=[
            pltpu.VMEM((_C5,), jnp.int32),
            pltpu.VMEM((_C5, 2304), jnp.float32),
            pltpu.VMEM((_C5, 640), jnp.float32),
            pltpu.VMEM((_C5, 16), jnp.float32),
            pltpu.VMEM((_C5, 9 * D), jnp.float32),
            pltpu.SemaphoreType.DMA,
        ],
    )
    def k(tj2_hbm, rs_hbm, aux_hbm, nj_hbm, upd_hbm,
          idx_j, tj2, rsr, auxr, updst, sem):
        wid = lax.axis_index("s") * 2 + lax.axis_index("c")

        def chunk_body(ch, carry):
            off = lax.min(ch * _C5, _EPW - _C5)
            base = wid * _EPW + off
            pltpu.sync_copy(nj_hbm.at[pl.ds(base, _C5)], idx_j)
            pltpu.async_copy(tj2_hbm.at[idx_j], tj2, sem).wait()
            pltpu.sync_copy(rs_hbm.at[pl.ds(base, _C5)], rsr)
            pltpu.sync_copy(aux_hbm.at[pl.ds(base, _C5)], auxr)

            def edge_body(e, c2):
                auxv = auxr[e, pl.ds(0, 16)]

                def o_group(g):
                    p_s = auxv[g // 5]
                    return (p_s * tj2[e, pl.ds(g * 16, 16)]
                            + rsr[e, pl.ds(g * 16, 16)]
                            * tj2[e, pl.ds(640 + g * 16, 16)])

                for g in range(8):
                    updst[e, pl.ds(g * 16, 16)] = o_group(g)
                c1 = [o_group(8 + g) for g in range(8)]
                c3 = [o_group(24 + g) for g in range(8)]
                for k3 in range(3):
                    r1k = auxv[8 + k3]
                    for g in range(8):
                        x0g = tj2[e, pl.ds(1280 + k3 * D + g * 16, 16)]
                        updst[e, pl.ds(D + k3 * D + g * 16, 16)] = (
                            c1[g] * r1k + c3[g] * x0g)
                c2v = [o_group(16 + g) for g in range(8)]
                c4 = [o_group(32 + g) for g in range(8)]
                for k5 in range(5):
                    r2k = auxv[11 + k5]
                    for g in range(8):
                        x1g = tj2[e, pl.ds(1664 + k5 * D + g * 16, 16)]
                        updst[e, pl.ds(4 * D + k5 * D + g * 16, 16)] = (
                            c2v[g] * r2k + c4[g] * x1g)
                return c2

            lax.fori_loop(0, _C5, edge_body, 0)
            pltpu.sync_copy(updst, upd_hbm.at[pl.ds(base, _C5)])
            return carry

        lax.fori_loop(0, _NCH5, chunk_body, 0)

    return k(Tj2, rs_cc, aux, nj_arr)


def _add_body(a_ref, b_ref, o_ref):
    o_ref[...] = a_ref[...] + b_ref[...]


def _pl_add(a, b):
    return pl.pallas_call(
        _add_body,
        out_shape=jax.ShapeDtypeStruct(a.shape, a.dtype),
    )(a, b)


def kernel(h, X_0, X_1, t_ij, r_ij_0, r_ij_1, r_ij_2, edge_index, params):
    n_j, n_i = edge_index[0], edge_index[1]
    X = [X_0, X_1]
    r = [r_ij_0, r_ij_1, r_ij_2]
    Xcat = jnp.concatenate(X, axis=1)
    eqf = (Xcat @ params['w_vq']).reshape(N, 8 * ERD)
    ekf = jnp.concatenate(
        [X[i] @ params['w_vk'][i] for i in range(DEG)], axis=1).reshape(N, 8 * ERD)
    hn = _layer_norm(h, params['ln_g'], params['ln_b'])
    q = hn @ params['w_q']
    kk = hn @ params['w_k']
    Ta = jnp.concatenate([eqf, q], axis=1)
    Tb = jnp.concatenate([ekf, kk], axis=1)
    w_ij, qk = _gather_wqk(Ta, Tb, n_i, n_j)
    dt_ij = _mlp(params['mlp_w'], w_ij) * _mlp(params['mlp_t'], t_ij)
    t_new = t_ij + dt_ij
    re = jax.nn.silu(t_new @ params['w_re'])
    a = (qk * re).reshape(E, H, -1).sum(axis=-1)
    p = _scatter_softmax(a, n_i, N)
    cc = _cos_cutoff(r[0])
    rs_cc = (t_new @ params['w_rs']) * cc
    V = _mlp(params['mlp_v'], hn)
    SH = _mlp(params['mlp_s'], h)
    Tj2 = jnp.concatenate([V, SH, X_0.reshape(N, 3 * D),
                           X_1.reshape(N, 5 * D)], axis=1)
    aux = jnp.concatenate([p, r[1], r[2]], axis=1)
    upd = _edge_update(Tj2, rs_cc, aux, n_j)
    agg = jax.ops.segment_sum(upd, n_i, num_segments=h.shape[0])
    h_new = _pl_add(h, agg[:, :D])
    X_new = [X[0] + agg[:, D:4 * D].reshape(N, 3, D),
             X[1] + agg[:, 4 * D:].reshape(N, 5, D)]
    return (h_new, X_new[0], X_new[1], t_new)


# speedup vs baseline: 6.3889x; 1.0692x over previous
import functools

import jax
import jax.numpy as jnp
from jax import lax
from jax.experimental import pallas as pl
from jax.experimental.pallas import tpu as pltpu
from jax.experimental.pallas import tpu_sc as plsc

N = 10000
E = 160000
D = 128
ERD = 64
DEG = 2
H = 8
S = 1 + 2 * DEG
CUT = 5.0


def _mlp(p, x):
    return jax.nn.silu(x @ p['W1'] + p['b1']) @ p['W2'] + p['b2']


def _layer_norm(x, g, b, eps=1e-5):
    mu = x.mean(-1, keepdims=True)
    var = ((x - mu) ** 2).mean(-1, keepdims=True)
    return (x - mu) / jnp.sqrt(var + eps) * g + b


def _cos_cutoff(d, c=CUT):
    mask = (d <= c).astype(d.dtype)
    return ((jnp.cos(jnp.pi * d / c) + 1.0) / 2.0) * mask


def _scatter_softmax(a, seg, num_segments):
    m = jax.ops.segment_max(a, seg, num_segments=num_segments)
    m = jnp.where(jnp.isfinite(m), m, 0.0)
    e = jnp.exp(a - m[seg])
    s = jax.ops.segment_sum(e, seg, num_segments=num_segments)
    return e / (s[seg] + 1e-16)


_NW = 32
_EPW = E // _NW
_C = 40
_NCH = _EPW // _C


def _gather_wqk(Ta, Tb, ni_arr, nj_arr):
    mesh = plsc.VectorSubcoreMesh(core_axis_name="c", subcore_axis_name="s")

    @functools.partial(
        pl.kernel,
        out_type=(jax.ShapeDtypeStruct((E, ERD), jnp.float32),
                  jax.ShapeDtypeStruct((E, D), jnp.float32)),
        mesh=mesh,
        scratch_types=[
            pltpu.VMEM((_C,), jnp.int32),
            pltpu.VMEM((_C,), jnp.int32),
            pltpu.VMEM((_C, 640), jnp.float32),
            pltpu.VMEM((_C, 640), jnp.float32),
            pltpu.VMEM((_C, ERD), jnp.float32),
            pltpu.VMEM((_C, D), jnp.float32),
            pltpu.SemaphoreType.DMA,
            pltpu.SemaphoreType.DMA,
        ],
    )
    def k(ta_hbm, tb_hbm, ni_hbm, nj_hbm, w_hbm, qk_hbm,
          idx_i, idx_j, arows, brows, wst, qkst, sem1, sem2):
        wid = lax.axis_index("s") * 2 + lax.axis_index("c")

        def chunk_body(ch, carry):
            base = wid * _EPW + ch * _C
            pltpu.sync_copy(ni_hbm.at[pl.ds(base, _C)], idx_i)
            pltpu.sync_copy(nj_hbm.at[pl.ds(base, _C)], idx_j)
            cp1 = pltpu.async_copy(ta_hbm.at[idx_i], arows, sem1)
            cp2 = pltpu.async_copy(tb_hbm.at[idx_j], brows, sem2)
            cp1.wait()
            cp2.wait()

            def edge_body(e, c2):
                for g in range(4):
                    acc = arows[e, pl.ds(g * 16, 16)] * brows[e, pl.ds(g * 16, 16)]
                    for s_ in range(1, 8):
                        off = s_ * ERD + g * 16
                        acc = acc + (arows[e, pl.ds(off, 16)]
                                     * brows[e, pl.ds(off, 16)])
                    wst[e, pl.ds(g * 16, 16)] = acc
                for g in range(8):
                    off = 512 + g * 16
                    qkst[e, pl.ds(g * 16, 16)] = (arows[e, pl.ds(off, 16)]
                                                  * brows[e, pl.ds(off, 16)])
                return c2

            lax.fori_loop(0, _C, edge_body, 0)
            pltpu.sync_copy(wst, w_hbm.at[pl.ds(base, _C)])
            pltpu.sync_copy(qkst, qk_hbm.at[pl.ds(base, _C)])
            return carry

        lax.fori_loop(0, _NCH, chunk_body, 0)

    return k(Ta, Tb, ni_arr, nj_arr)


_C5 = 16
_NCH5 = 2 * (-(-_EPW // (2 * _C5)))


def _edge_update(Tj2, RA, nj_arr):
    mesh = plsc.VectorSubcoreMesh(core_axis_name="c", subcore_axis_name="s")

    @functools.partial(
        pl.kernel,
        out_type=jax.ShapeDtypeStruct((E, 9 * D), jnp.float32),
        mesh=mesh,
        scratch_types=[
            [pltpu.VMEM((_C5,), jnp.int32) for _ in range(2)],
            [pltpu.VMEM((_C5, 2304), jnp.float32) for _ in range(2)],
            [pltpu.VMEM((_C5, 656), jnp.float32) for _ in range(2)],
            pltpu.VMEM((_C5, 9 * D), jnp.float32),
            [pltpu.SemaphoreType.DMA for _ in range(2)],
            [pltpu.SemaphoreType.DMA for _ in range(2)],
        ],
    )
    def k(tj2_hbm, ra_hbm, nj_hbm, upd_hbm,
          idx_j, tj2, ra, updst, semg, semr):
        wid = lax.axis_index("s") * 2 + lax.axis_index("c")

        def chbase(ch):
            return wid * _EPW + lax.min(ch * _C5, _EPW - _C5)

        def start(ch, b):
            base = chbase(ch)
            pltpu.sync_copy(nj_hbm.at[pl.ds(base, _C5)], idx_j[b])
            pltpu.async_copy(tj2_hbm.at[idx_j[b]], tj2[b], semg[b])
            pltpu.async_copy(ra_hbm.at[pl.ds(base, _C5)], ra[b], semr[b])

        def compute(ch, b):
            base = chbase(ch)
            pltpu.make_async_copy(tj2_hbm.at[idx_j[b]], tj2[b], semg[b]).wait()
            pltpu.make_async_copy(ra_hbm.at[pl.ds(base, _C5)], ra[b],
                                  semr[b]).wait()

            def edge_body(e, c2):
                auxv = ra[b][e, pl.ds(640, 16)]

                def o_group(g):
                    p_s = auxv[g // 5]
                    return (p_s * tj2[b][e, pl.ds(g * 16, 16)]
                            + ra[b][e, pl.ds(g * 16, 16)]
                            * tj2[b][e, pl.ds(640 + g * 16, 16)])

                for g in range(8):
                    updst[e, pl.ds(g * 16, 16)] = o_group(g)
                c1 = [o_group(8 + g) for g in range(8)]
                c3 = [o_group(24 + g) for g in range(8)]
                for k3 in range(3):
                    r1k = auxv[8 + k3]
                    for g in range(8):
                        x0g = tj2[b][e, pl.ds(1280 + k3 * D + g * 16, 16)]
                        updst[e, pl.ds(D + k3 * D + g * 16, 16)] = (
                            c1[g] * r1k + c3[g] * x0g)
                c2v = [o_group(16 + g) for g in range(8)]
                c4 = [o_group(32 + g) for g in range(8)]
                for k5 in range(5):
                    r2k = auxv[11 + k5]
                    for g in range(8):
                        x1g = tj2[b][e, pl.ds(1664 + k5 * D + g * 16, 16)]
                        updst[e, pl.ds(4 * D + k5 * D + g * 16, 16)] = (
                            c2v[g] * r2k + c4[g] * x1g)
                return c2

            lax.fori_loop(0, _C5, edge_body, 0)
            pltpu.sync_copy(updst, upd_hbm.at[pl.ds(base, _C5)])

        start(0, 0)
        start(1, 1)

        def pair_body(i2, carry):
            cha = 2 * i2
            compute(cha, 0)

            @pl.when(cha + 2 < _NCH5)
            def _():
                start(cha + 2, 0)

            compute(cha + 1, 1)

            @pl.when(cha + 3 < _NCH5)
            def _():
                start(cha + 3, 1)

            return carry

        lax.fori_loop(0, _NCH5 // 2, pair_body, 0)

    return k(Tj2, RA, nj_arr)


def _add_body(a_ref, b_ref, o_ref):
    o_ref[...] = a_ref[...] + b_ref[...]


def _pl_add(a, b):
    return pl.pallas_call(
        _add_body,
        out_shape=jax.ShapeDtypeStruct(a.shape, a.dtype),
    )(a, b)


def kernel(h, X_0, X_1, t_ij, r_ij_0, r_ij_1, r_ij_2, edge_index, params):
    n_j, n_i = edge_index[0], edge_index[1]
    X = [X_0, X_1]
    r = [r_ij_0, r_ij_1, r_ij_2]
    Xcat = jnp.concatenate(X, axis=1)
    eqf = (Xcat @ params['w_vq']).reshape(N, 8 * ERD)
    ekf = jnp.concatenate(
        [X[i] @ params['w_vk'][i] for i in range(DEG)], axis=1).reshape(N, 8 * ERD)
    hn = _layer_norm(h, params['ln_g'], params['ln_b'])
    q = hn @ params['w_q']
    kk = hn @ params['w_k']
    Ta = jnp.concatenate([eqf, q], axis=1)
    Tb = jnp.concatenate([ekf, kk], axis=1)
    w_ij, qk = _gather_wqk(Ta, Tb, n_i, n_j)
    dt_ij = _mlp(params['mlp_w'], w_ij) * _mlp(params['mlp_t'], t_ij)
    t_new = t_ij + dt_ij
    re = jax.nn.silu(t_new @ params['w_re'])
    a = (qk * re).reshape(E, H, -1).sum(axis=-1)
    p = _scatter_softmax(a, n_i, N)
    cc = _cos_cutoff(r[0])
    rs_cc = (t_new @ params['w_rs']) * cc
    V = _mlp(params['mlp_v'], hn)
    SH = _mlp(params['mlp_s'], h)
    Tj2 = jnp.concatenate([V, SH, X_0.reshape(N, 3 * D),
                           X_1.reshape(N, 5 * D)], axis=1)
    RA = jnp.concatenate([rs_cc, p, r[1], r[2]], axis=1)
    upd = _edge_update(Tj2, RA, n_j)
    agg = jax.ops.segment_sum(upd, n_i, num_segments=h.shape[0])
    h_new = _pl_add(h, agg[:, :D])
    X_new = [X[0] + agg[:, D:4 * D].reshape(N, 3, D),
             X[1] + agg[:, 4 * D:].reshape(N, 5, D)]
    return (h_new, X_new[0], X_new[1], t_new)


# speedup vs baseline: 6.5820x; 1.0302x over previous
import functools

import jax
import jax.numpy as jnp
from jax import lax
from jax.experimental import pallas as pl
from jax.experimental.pallas import tpu as pltpu
from jax.experimental.pallas import tpu_sc as plsc

N = 10000
E = 160000
D = 128
ERD = 64
DEG = 2
H = 8
S = 1 + 2 * DEG
CUT = 5.0


def _mlp(p, x):
    return jax.nn.silu(x @ p['W1'] + p['b1']) @ p['W2'] + p['b2']


def _layer_norm(x, g, b, eps=1e-5):
    mu = x.mean(-1, keepdims=True)
    var = ((x - mu) ** 2).mean(-1, keepdims=True)
    return (x - mu) / jnp.sqrt(var + eps) * g + b


def _cos_cutoff(d, c=CUT):
    mask = (d <= c).astype(d.dtype)
    return ((jnp.cos(jnp.pi * d / c) + 1.0) / 2.0) * mask


def _scatter_softmax(a, seg, num_segments):
    m = jax.ops.segment_max(a, seg, num_segments=num_segments)
    m = jnp.where(jnp.isfinite(m), m, 0.0)
    e = jnp.exp(a - m[seg])
    s = jax.ops.segment_sum(e, seg, num_segments=num_segments)
    return e / (s[seg] + 1e-16)


_NW = 32
_EPW = E // _NW
_C = 40
_NCH = _EPW // _C


def _gather_wqk(Ta, Tb, ni_arr, nj_arr):
    mesh = plsc.VectorSubcoreMesh(core_axis_name="c", subcore_axis_name="s")

    @functools.partial(
        pl.kernel,
        out_type=(jax.ShapeDtypeStruct((E, ERD), jnp.float32),
                  jax.ShapeDtypeStruct((E, D), jnp.float32)),
        mesh=mesh,
        scratch_types=[
            pltpu.VMEM((_EPW,), jnp.int32),
            pltpu.VMEM((_EPW,), jnp.int32),
            pltpu.VMEM((_C, 640), jnp.float32),
            pltpu.VMEM((_C, 640), jnp.float32),
            pltpu.VMEM((_C, ERD), jnp.float32),
            pltpu.VMEM((_C, D), jnp.float32),
            pltpu.SemaphoreType.DMA,
            pltpu.SemaphoreType.DMA,
        ],
    )
    def k(ta_hbm, tb_hbm, ni_hbm, nj_hbm, w_hbm, qk_hbm,
          idx_i, idx_j, arows, brows, wst, qkst, sem1, sem2):
        wid = lax.axis_index("s") * 2 + lax.axis_index("c")
        pltpu.sync_copy(ni_hbm.at[pl.ds(wid * _EPW, _EPW)], idx_i)
        pltpu.sync_copy(nj_hbm.at[pl.ds(wid * _EPW, _EPW)], idx_j)

        def chunk_body(ch, carry):
            off = ch * _C
            base = wid * _EPW + off
            cp1 = pltpu.async_copy(ta_hbm.at[idx_i.at[pl.ds(off, _C)]],
                                   arows, sem1)
            cp2 = pltpu.async_copy(tb_hbm.at[idx_j.at[pl.ds(off, _C)]],
                                   brows, sem2)
            cp1.wait()
            cp2.wait()

            def edge_body(e, c2):
                for g in range(4):
                    acc = arows[e, pl.ds(g * 16, 16)] * brows[e, pl.ds(g * 16, 16)]
                    for s_ in range(1, 8):
                        off = s_ * ERD + g * 16
                        acc = acc + (arows[e, pl.ds(off, 16)]
                                     * brows[e, pl.ds(off, 16)])
                    wst[e, pl.ds(g * 16, 16)] = acc
                for g in range(8):
                    off = 512 + g * 16
                    qkst[e, pl.ds(g * 16, 16)] = (arows[e, pl.ds(off, 16)]
                                                  * brows[e, pl.ds(off, 16)])
                return c2

            lax.fori_loop(0, _C, edge_body, 0)
            pltpu.sync_copy(wst, w_hbm.at[pl.ds(base, _C)])
            pltpu.sync_copy(qkst, qk_hbm.at[pl.ds(base, _C)])
            return carry

        lax.fori_loop(0, _NCH, chunk_body, 0)

    return k(Ta, Tb, ni_arr, nj_arr)


_C5 = 16
_NCH5 = 2 * (-(-_EPW // (2 * _C5)))


def _edge_update(Tj2, RA, nj_arr):
    mesh = plsc.VectorSubcoreMesh(core_axis_name="c", subcore_axis_name="s")

    @functools.partial(
        pl.kernel,
        out_type=jax.ShapeDtypeStruct((E, 9 * D), jnp.float32),
        mesh=mesh,
        scratch_types=[
            pltpu.VMEM((_EPW,), jnp.int32),
            [pltpu.VMEM((_C5, 2304), jnp.float32) for _ in range(2)],
            [pltpu.VMEM((_C5, 656), jnp.float32) for _ in range(2)],
            pltpu.VMEM((_C5, 9 * D), jnp.float32),
            [pltpu.SemaphoreType.DMA for _ in range(2)],
            [pltpu.SemaphoreType.DMA for _ in range(2)],
        ],
    )
    def k(tj2_hbm, ra_hbm, nj_hbm, upd_hbm,
          idx_j, tj2, ra, updst, semg, semr):
        wid = lax.axis_index("s") * 2 + lax.axis_index("c")
        pltpu.sync_copy(nj_hbm.at[pl.ds(wid * _EPW, _EPW)], idx_j)

        def choff(ch):
            return lax.min(ch * _C5, _EPW - _C5)

        def chbase(ch):
            return wid * _EPW + choff(ch)

        def start(ch, b):
            base = chbase(ch)
            pltpu.async_copy(tj2_hbm.at[idx_j.at[pl.ds(choff(ch), _C5)]],
                             tj2[b], semg[b])
            pltpu.async_copy(ra_hbm.at[pl.ds(base, _C5)], ra[b], semr[b])

        def compute(ch, b):
            base = chbase(ch)
            pltpu.make_async_copy(tj2_hbm.at[idx_j.at[pl.ds(choff(ch), _C5)]],
                                  tj2[b], semg[b]).wait()
            pltpu.make_async_copy(ra_hbm.at[pl.ds(base, _C5)], ra[b],
                                  semr[b]).wait()

            def edge_body(e, c2):
                auxv = ra[b][e, pl.ds(640, 16)]

                def o_group(g):
                    p_s = auxv[g // 5]
                    return (p_s * tj2[b][e, pl.ds(g * 16, 16)]
                            + ra[b][e, pl.ds(g * 16, 16)]
                            * tj2[b][e, pl.ds(640 + g * 16, 16)])

                for g in range(8):
                    updst[e, pl.ds(g * 16, 16)] = o_group(g)
                c1 = [o_group(8 + g) for g in range(8)]
                c3 = [o_group(24 + g) for g in range(8)]
                for k3 in range(3):
                    r1k = auxv[8 + k3]
                    for g in range(8):
                        x0g = tj2[b][e, pl.ds(1280 + k3 * D + g * 16, 16)]
                        updst[e, pl.ds(D + k3 * D + g * 16, 16)] = (
                            c1[g] * r1k + c3[g] * x0g)
                c2v = [o_group(16 + g) for g in range(8)]
                c4 = [o_group(32 + g) for g in range(8)]
                for k5 in range(5):
                    r2k = auxv[11 + k5]
                    for g in range(8):
                        x1g = tj2[b][e, pl.ds(1664 + k5 * D + g * 16, 16)]
                        updst[e, pl.ds(4 * D + k5 * D + g * 16, 16)] = (
                            c2v[g] * r2k + c4[g] * x1g)
                return c2

            lax.fori_loop(0, _C5, edge_body, 0)
            pltpu.sync_copy(updst, upd_hbm.at[pl.ds(base, _C5)])

        start(0, 0)
        start(1, 1)

        def pair_body(i2, carry):
            cha = 2 * i2
            compute(cha, 0)

            @pl.when(cha + 2 < _NCH5)
            def _():
                start(cha + 2, 0)

            compute(cha + 1, 1)

            @pl.when(cha + 3 < _NCH5)
            def _():
                start(cha + 3, 1)

            return carry

        lax.fori_loop(0, _NCH5 // 2, pair_body, 0)

    return k(Tj2, RA, nj_arr)


def _add_body(a_ref, b_ref, o_ref):
    o_ref[...] = a_ref[...] + b_ref[...]


def _pl_add(a, b):
    return pl.pallas_call(
        _add_body,
        out_shape=jax.ShapeDtypeStruct(a.shape, a.dtype),
    )(a, b)


def kernel(h, X_0, X_1, t_ij, r_ij_0, r_ij_1, r_ij_2, edge_index, params):
    n_j, n_i = edge_index[0], edge_index[1]
    X = [X_0, X_1]
    r = [r_ij_0, r_ij_1, r_ij_2]
    Xcat = jnp.concatenate(X, axis=1)
    eqf = (Xcat @ params['w_vq']).reshape(N, 8 * ERD)
    ekf = jnp.concatenate(
        [X[i] @ params['w_vk'][i] for i in range(DEG)], axis=1).reshape(N, 8 * ERD)
    hn = _layer_norm(h, params['ln_g'], params['ln_b'])
    q = hn @ params['w_q']
    kk = hn @ params['w_k']
    Ta = jnp.concatenate([eqf, q], axis=1)
    Tb = jnp.concatenate([ekf, kk], axis=1)
    w_ij, qk = _gather_wqk(Ta, Tb, n_i, n_j)
    dt_ij = _mlp(params['mlp_w'], w_ij) * _mlp(params['mlp_t'], t_ij)
    t_new = t_ij + dt_ij
    re = jax.nn.silu(t_new @ params['w_re'])
    a = (qk * re).reshape(E, H, -1).sum(axis=-1)
    p = _scatter_softmax(a, n_i, N)
    cc = _cos_cutoff(r[0])
    rs_cc = (t_new @ params['w_rs']) * cc
    V = _mlp(params['mlp_v'], hn)
    SH = _mlp(params['mlp_s'], h)
    Tj2 = jnp.concatenate([V, SH, X_0.reshape(N, 3 * D),
                           X_1.reshape(N, 5 * D)], axis=1)
    RA = jnp.concatenate([rs_cc, p, r[1], r[2]], axis=1)
    upd = _edge_update(Tj2, RA, n_j)
    agg = jax.ops.segment_sum(upd, n_i, num_segments=h.shape[0])
    h_new = _pl_add(h, agg[:, :D])
    X_new = [X[0] + agg[:, D:4 * D].reshape(N, 3, D),
             X[1] + agg[:, 4 * D:].reshape(N, 5, D)]
    return (h_new, X_new[0], X_new[1], t_new)


# speedup vs baseline: 6.7477x; 1.0252x over previous
import functools

import jax
import jax.numpy as jnp
from jax import lax
from jax.experimental import pallas as pl
from jax.experimental.pallas import tpu as pltpu
from jax.experimental.pallas import tpu_sc as plsc

N = 10000
E = 160000
D = 128
ERD = 64
DEG = 2
H = 8
S = 1 + 2 * DEG
CUT = 5.0


def _mlp(p, x):
    return jax.nn.silu(x @ p['W1'] + p['b1']) @ p['W2'] + p['b2']


def _layer_norm(x, g, b, eps=1e-5):
    mu = x.mean(-1, keepdims=True)
    var = ((x - mu) ** 2).mean(-1, keepdims=True)
    return (x - mu) / jnp.sqrt(var + eps) * g + b


def _cos_cutoff(d, c=CUT):
    mask = (d <= c).astype(d.dtype)
    return ((jnp.cos(jnp.pi * d / c) + 1.0) / 2.0) * mask


def _scatter_softmax(a, seg, num_segments):
    m = jax.ops.segment_max(a, seg, num_segments=num_segments)
    m = jnp.where(jnp.isfinite(m), m, 0.0)
    e = jnp.exp(a - m[seg])
    s = jax.ops.segment_sum(e, seg, num_segments=num_segments)
    return e / (s[seg] + 1e-16)


_NW = 32
_EPW = E // _NW
_C = 40
_NCH = _EPW // _C


def _gather_wqk(Ta, Tb, ni_arr, nj_arr):
    mesh = plsc.VectorSubcoreMesh(core_axis_name="c", subcore_axis_name="s")

    @functools.partial(
        pl.kernel,
        out_type=(jax.ShapeDtypeStruct((E, ERD), jnp.float32),
                  jax.ShapeDtypeStruct((E, D), jnp.float32)),
        mesh=mesh,
        scratch_types=[
            pltpu.VMEM((_EPW,), jnp.int32),
            pltpu.VMEM((_EPW,), jnp.int32),
            pltpu.VMEM((_C, 640), jnp.float32),
            pltpu.VMEM((_C, 640), jnp.float32),
            pltpu.VMEM((_C, ERD), jnp.float32),
            pltpu.VMEM((_C, D), jnp.float32),
            pltpu.SemaphoreType.DMA,
            pltpu.SemaphoreType.DMA,
        ],
    )
    def k(ta_hbm, tb_hbm, ni_hbm, nj_hbm, w_hbm, qk_hbm,
          idx_i, idx_j, arows, brows, wst, qkst, sem1, sem2):
        wid = lax.axis_index("s") * 2 + lax.axis_index("c")
        pltpu.sync_copy(ni_hbm.at[pl.ds(wid * _EPW, _EPW)], idx_i)
        pltpu.sync_copy(nj_hbm.at[pl.ds(wid * _EPW, _EPW)], idx_j)

        def chunk_body(ch, carry):
            off = ch * _C
            base = wid * _EPW + off
            cp1 = pltpu.async_copy(ta_hbm.at[idx_i.at[pl.ds(off, _C)]],
                                   arows, sem1)
            cp2 = pltpu.async_copy(tb_hbm.at[idx_j.at[pl.ds(off, _C)]],
                                   brows, sem2)
            cp1.wait()
            cp2.wait()

            def edge_body(e, c2):
                for g in range(4):
                    acc = arows[e, pl.ds(g * 16, 16)] * brows[e, pl.ds(g * 16, 16)]
                    for s_ in range(1, 8):
                        off = s_ * ERD + g * 16
                        acc = acc + (arows[e, pl.ds(off, 16)]
                                     * brows[e, pl.ds(off, 16)])
                    wst[e, pl.ds(g * 16, 16)] = acc
                for g in range(8):
                    off = 512 + g * 16
                    qkst[e, pl.ds(g * 16, 16)] = (arows[e, pl.ds(off, 16)]
                                                  * brows[e, pl.ds(off, 16)])
                return c2

            lax.fori_loop(0, _C, edge_body, 0)
            pltpu.sync_copy(wst, w_hbm.at[pl.ds(base, _C)])
            pltpu.sync_copy(qkst, qk_hbm.at[pl.ds(base, _C)])
            return carry

        lax.fori_loop(0, _NCH, chunk_body, 0)

    return k(Ta, Tb, ni_arr, nj_arr)


_C5 = 16
_NCH5 = 2 * (-(-_EPW // (2 * _C5)))


def _edge_update(Tj2, RA, nj_arr):
    mesh = plsc.VectorSubcoreMesh(core_axis_name="c", subcore_axis_name="s")

    @functools.partial(
        pl.kernel,
        out_type=jax.ShapeDtypeStruct((E, 9 * D), jnp.float32),
        mesh=mesh,
        scratch_types=[
            pltpu.VMEM((_EPW,), jnp.int32),
            [pltpu.VMEM((_C5, 2304), jnp.float32) for _ in range(2)],
            [pltpu.VMEM((_C5, 656), jnp.float32) for _ in range(2)],
            pltpu.VMEM((_C5, 9 * D), jnp.float32),
            [pltpu.SemaphoreType.DMA for _ in range(2)],
            [pltpu.SemaphoreType.DMA for _ in range(2)],
        ],
    )
    def k(tj2_hbm, ra_hbm, nj_hbm, upd_hbm,
          idx_j, tj2, ra, updst, semg, semr):
        wid = lax.axis_index("s") * 2 + lax.axis_index("c")
        pltpu.sync_copy(nj_hbm.at[pl.ds(wid * _EPW, _EPW)], idx_j)

        def choff(ch):
            return lax.min(ch * _C5, _EPW - _C5)

        def chbase(ch):
            return wid * _EPW + choff(ch)

        def start(ch, b):
            base = chbase(ch)
            pltpu.async_copy(tj2_hbm.at[idx_j.at[pl.ds(choff(ch), _C5)]],
                             tj2[b], semg[b])
            pltpu.async_copy(ra_hbm.at[pl.ds(base, _C5)], ra[b], semr[b])

        def compute(ch, b):
            base = chbase(ch)
            pltpu.make_async_copy(tj2_hbm.at[idx_j.at[pl.ds(choff(ch), _C5)]],
                                  tj2[b], semg[b]).wait()
            pltpu.make_async_copy(ra_hbm.at[pl.ds(base, _C5)], ra[b],
                                  semr[b]).wait()

            def edge_body(e, c2):
                auxv = ra[b][e, pl.ds(640, 16)]

                def o_group(g):
                    p_s = auxv[g // 5]
                    return (p_s * tj2[b][e, pl.ds(g * 16, 16)]
                            + ra[b][e, pl.ds(g * 16, 16)]
                            * tj2[b][e, pl.ds(640 + g * 16, 16)])

                for g in range(8):
                    updst[e, pl.ds(g * 16, 16)] = o_group(g)
                c1 = [o_group(8 + g) for g in range(8)]
                c3 = [o_group(24 + g) for g in range(8)]
                for k3 in range(3):
                    r1k = auxv[8 + k3]
                    for g in range(8):
                        x0g = tj2[b][e, pl.ds(1280 + k3 * D + g * 16, 16)]
                        updst[e, pl.ds(D + k3 * D + g * 16, 16)] = (
                            c1[g] * r1k + c3[g] * x0g)
                c2v = [o_group(16 + g) for g in range(8)]
                c4 = [o_group(32 + g) for g in range(8)]
                for k5 in range(5):
                    r2k = auxv[11 + k5]
                    for g in range(8):
                        x1g = tj2[b][e, pl.ds(1664 + k5 * D + g * 16, 16)]
                        updst[e, pl.ds(4 * D + k5 * D + g * 16, 16)] = (
                            c2v[g] * r2k + c4[g] * x1g)
                return c2

            lax.fori_loop(0, _C5, edge_body, 0)
            pltpu.sync_copy(updst, upd_hbm.at[pl.ds(base, _C5)])

        start(0, 0)
        start(1, 1)

        def pair_body(i2, carry):
            cha = 2 * i2
            compute(cha, 0)

            @pl.when(cha + 2 < _NCH5)
            def _():
                start(cha + 2, 0)

            compute(cha + 1, 1)

            @pl.when(cha + 3 < _NCH5)
            def _():
                start(cha + 3, 1)

            return carry

        lax.fori_loop(0, _NCH5 // 2, pair_body, 0)

    return k(Tj2, RA, nj_arr)


_BE = 1000


def _edge_dense_body(t_ref, w_ref, qk_ref, cc_ref, sel_ref,
                     w1w, b1w, w2w, b2w, w1t, b1t, w2t, b2t, wre, wrs,
                     tn_ref, a_ref, rs_ref):
    t = t_ref[...]
    hw = jax.nn.silu(w_ref[...] @ w1w[...] + b1w[...]) @ w2w[...] + b2w[...]
    ht = jax.nn.silu(t @ w1t[...] + b1t[...]) @ w2t[...] + b2t[...]
    tn = t + hw * ht
    tn_ref[...] = tn
    re = jax.nn.silu(tn @ wre[...])
    a_ref[...] = (qk_ref[...] * re) @ sel_ref[...]
    rs_ref[...] = (tn @ wrs[...]) * cc_ref[...]


def _edge_dense(t_ij, w_ij, qk, cc, params):
    sel = jnp.repeat(jnp.eye(H, dtype=jnp.float32), 16, axis=0)
    pw, pt = params['mlp_w'], params['mlp_t']
    row = lambda b: b.reshape(1, -1)
    grid = E // _BE
    eb = lambda width: pl.BlockSpec((_BE, width), lambda i: (i, 0))
    full = lambda a: pl.BlockSpec(a.shape, lambda i: (0,) * a.ndim)
    args = (t_ij, w_ij, qk, cc, sel,
            pw['W1'], row(pw['b1']), pw['W2'], row(pw['b2']),
            pt['W1'], row(pt['b1']), pt['W2'], row(pt['b2']),
            params['w_re'], params['w_rs'])
    in_specs = [eb(D), eb(ERD), eb(D), eb(1)] + [full(a) for a in args[4:]]
    return pl.pallas_call(
        _edge_dense_body,
        grid=(grid,),
        in_specs=in_specs,
        out_specs=[eb(D), eb(H), eb(S * D)],
        out_shape=[jax.ShapeDtypeStruct((E, D), jnp.float32),
                   jax.ShapeDtypeStruct((E, H), jnp.float32),
                   jax.ShapeDtypeStruct((E, S * D), jnp.float32)],
    )(*args)


def _add_body(a_ref, b_ref, o_ref):
    o_ref[...] = a_ref[...] + b_ref[...]


def _pl_add(a, b):
    return pl.pallas_call(
        _add_body,
        out_shape=jax.ShapeDtypeStruct(a.shape, a.dtype),
    )(a, b)


def kernel(h, X_0, X_1, t_ij, r_ij_0, r_ij_1, r_ij_2, edge_index, params):
    n_j, n_i = edge_index[0], edge_index[1]
    X = [X_0, X_1]
    r = [r_ij_0, r_ij_1, r_ij_2]
    Xcat = jnp.concatenate(X, axis=1)
    eqf = (Xcat @ params['w_vq']).reshape(N, 8 * ERD)
    ekf = jnp.concatenate(
        [X[i] @ params['w_vk'][i] for i in range(DEG)], axis=1).reshape(N, 8 * ERD)
    hn = _layer_norm(h, params['ln_g'], params['ln_b'])
    q = hn @ params['w_q']
    kk = hn @ params['w_k']
    Ta = jnp.concatenate([eqf, q], axis=1)
    Tb = jnp.concatenate([ekf, kk], axis=1)
    w_ij, qk = _gather_wqk(Ta, Tb, n_i, n_j)
    cc = _cos_cutoff(r[0])
    t_new, a, rs_cc = _edge_dense(t_ij, w_ij, qk, cc, params)
    p = _scatter_softmax(a, n_i, N)
    V = _mlp(params['mlp_v'], hn)
    SH = _mlp(params['mlp_s'], h)
    Tj2 = jnp.concatenate([V, SH, X_0.reshape(N, 3 * D),
                           X_1.reshape(N, 5 * D)], axis=1)
    RA = jnp.concatenate([rs_cc, p, r[1], r[2]], axis=1)
    upd = _edge_update(Tj2, RA, n_j)
    agg = jax.ops.segment_sum(upd, n_i, num_segments=h.shape[0])
    h_new = _pl_add(h, agg[:, :D])
    X_new = [X[0] + agg[:, D:4 * D].reshape(N, 3, D),
             X[1] + agg[:, 4 * D:].reshape(N, 5, D)]
    return (h_new, X_new[0], X_new[1], t_new)
